# Initial kernel scaffold; baseline (speedup 1.0000x reference)
import jax
import jax.numpy as jnp
from jax.experimental import pallas as pl


def kernel(features, paths_indices, kernel_size, weight):
    raise NotImplementedError("write your pallas kernel here")


# speedup vs baseline: 1.2670x; 1.2670x over previous
import functools

import jax
import jax.numpy as jnp
from jax import lax
from jax.experimental import pallas as pl
from jax.experimental.pallas import tpu as pltpu
from jax.experimental.pallas import tpu_sc as plsc

EPS = 1e-4
ALPHA = 4.0
N = 50000
NP = 800000
PS = 3
D = 128
H = 32

NW = 32
CH = 128
CHUNKS = 196
PPW = CH * CHUNKS
NPP = NW * PPW
N_POOL = 50048
RPT = N_POOL // 16
BN = 2000
NB = N // BN


def _tc1_body(f_ref, w_ref, e_ref, lt_ref):
    w2 = w_ref[...].reshape(PS * D, H)
    colnorm = jnp.maximum(jnp.sqrt(jnp.sum(w2 * w2, axis=0, keepdims=True)), EPS)
    wn = w2 / colnorm
    f = f_ref[...]
    inv = lax.rsqrt(jnp.maximum(jnp.sum(f * f, axis=1, keepdims=True), EPS * EPS))
    fn = f * inv
    wj = wn.reshape(PS, D, H)
    for j in range(PS):
        e_ref[j] = jnp.dot(fn, wj[j], preferred_element_type=jnp.float32) * (1.0 / PS)

    @pl.when(pl.program_id(0) == 0)
    def _():
        gram = jnp.exp(
            ALPHA * (lax.dot_general(wn, wn, (((0,), (0,)), ((), ())),
                                     preferred_element_type=jnp.float32) - 1.0))
        c = jnp.sqrt(jnp.sum(gram * gram))
        eye = jnp.eye(H, dtype=jnp.float32)
        y = gram * (1.0 / c)
        z = eye
        for _ in range(24):
            t = 1.5 * eye - 0.5 * jnp.dot(z, y, preferred_element_type=jnp.float32)
            y = jnp.dot(y, t, preferred_element_type=jnp.float32)
            z = jnp.dot(t, z, preferred_element_type=jnp.float32)
        lt_ref[...] = z * lax.rsqrt(c)


_tc1 = pl.pallas_call(
    _tc1_body,
    grid=(NB,),
    in_specs=[
        pl.BlockSpec((BN, D), lambda i: (i, 0)),
        pl.BlockSpec((PS, D, H), lambda i: (0, 0, 0)),
    ],
    out_specs=[
        pl.BlockSpec((PS, BN, H), lambda i: (0, i, 0)),
        pl.BlockSpec((H, H), lambda i: (0, 0)),
    ],
    out_shape=[
        jax.ShapeDtypeStruct((PS, N, H), jnp.float32),
        jax.ShapeDtypeStruct((H, H), jnp.float32),
    ],
)


def _sc_body(e_hbm, g0_hbm, g1_hbm, g2_hbm, seg_hbm, out_hbm,
             idx0, idx1, idx2, segv, r0, r1, r2, pooled, sem):
    c = lax.axis_index("c")
    s = lax.axis_index("s")
    wid = s * 2 + c

    def zero_row(p, _):
        r0[p, pl.ds(0, 16)] = jnp.zeros((16,), jnp.float32)
        r0[p, pl.ds(16, 16)] = jnp.zeros((16,), jnp.float32)
        return 0

    lax.fori_loop(0, CH, zero_row, 0)

    def zero_pool(i, _):
        pltpu.sync_copy(r0, pooled.at[pl.ds(s * RPT + i * CH, CH)])
        return 0

    lax.fori_loop(0, RPT // CH, zero_pool, 0)
    pltpu.sync_copy(r0.at[pl.ds(0, RPT - (RPT // CH) * CH)],
                    pooled.at[pl.ds(s * RPT + (RPT // CH) * CH,
                                    RPT - (RPT // CH) * CH)])
    plsc.subcore_barrier()

    def chunk(k, _):
        base = wid * PPW + k * CH
        pltpu.sync_copy(g0_hbm.at[pl.ds(base, CH)], idx0)
        pltpu.sync_copy(g1_hbm.at[pl.ds(base, CH)], idx1)
        pltpu.sync_copy(g2_hbm.at[pl.ds(base, CH)], idx2)
        pltpu.sync_copy(seg_hbm.at[pl.ds(base, CH)], segv)
        cp0 = pltpu.async_copy(e_hbm.at[idx0], r0, sem)
        cp1 = pltpu.async_copy(e_hbm.at[idx1], r1, sem)
        cp2 = pltpu.async_copy(e_hbm.at[idx2], r2, sem)
        cp0.wait()
        cp1.wait()
        cp2.wait()

        def compute(p, _):
            for h in (0, 16):
                v = r0[p, pl.ds(h, 16)] + r1[p, pl.ds(h, 16)] + r2[p, pl.ds(h, 16)]
                r0[p, pl.ds(h, 16)] = jnp.exp(ALPHA * v - ALPHA)
            return 0

        lax.fori_loop(0, CH, compute, 0)
        pltpu.sync_copy(r0, pooled.at[segv], add=True)
        return 0

    lax.fori_loop(0, CHUNKS, chunk, 0)
    plsc.subcore_barrier()
    pltpu.sync_copy(pooled.at[pl.ds(s * RPT, RPT)],
                    out_hbm.at[c, pl.ds(s * RPT, RPT)])


_sc_pool = functools.partial(
    pl.kernel,
    mesh=plsc.VectorSubcoreMesh(core_axis_name="c", subcore_axis_name="s"),
    compiler_params=pltpu.CompilerParams(use_tc_tiling_on_sc=False),
    out_type=jax.ShapeDtypeStruct((2, N_POOL, H), jnp.float32),
    scratch_types=[
        pltpu.VMEM((CH,), jnp.int32),
        pltpu.VMEM((CH,), jnp.int32),
        pltpu.VMEM((CH,), jnp.int32),
        pltpu.VMEM((CH,), jnp.int32),
        pltpu.VMEM((CH, H), jnp.float32),
        pltpu.VMEM((CH, H), jnp.float32),
        pltpu.VMEM((CH, H), jnp.float32),
        pltpu.VMEM_SHARED((N_POOL, H), jnp.float32),
        pltpu.SemaphoreType.DMA,
    ],
)(_sc_body)


def _tc2_body(p_ref, ks_ref, lt_ref, o_ref):
    p = p_ref[0] + p_ref[1]
    cnt = jnp.maximum(ks_ref[...], 1).astype(jnp.float32)
    o_ref[...] = jnp.dot(p / cnt, lt_ref[...], preferred_element_type=jnp.float32)


_tc2 = pl.pallas_call(
    _tc2_body,
    grid=(NB,),
    in_specs=[
        pl.BlockSpec((2, BN, H), lambda i: (0, i, 0)),
        pl.BlockSpec((BN, 1), lambda i: (i, 0)),
        pl.BlockSpec((H, H), lambda i: (0, 0)),
    ],
    out_specs=pl.BlockSpec((BN, H), lambda i: (i, 0)),
    out_shape=jax.ShapeDtypeStruct((N, H), jnp.float32),
)


@jax.jit
def kernel(features, paths_indices, kernel_size, weight):
    e3, lintrans = _tc1(features, weight)
    e_flat = e3.reshape(PS * N, H)

    cum = jnp.cumsum(kernel_size)
    seg = jnp.searchsorted(cum, jnp.arange(NP, dtype=jnp.int32),
                           side='right').astype(jnp.int32)
    segp = jnp.concatenate([seg, jnp.full((NPP - NP,), N, jnp.int32)])
    pad = jnp.zeros((NPP - NP,), jnp.int32)
    gs = [jnp.concatenate([paths_indices[:, j] + j * N, pad]) for j in range(PS)]

    pooled2 = _sc_pool(e_flat, gs[0], gs[1], gs[2], segp)[:, :N, :]
    return _tc2(pooled2, kernel_size.reshape(N, 1), lintrans)


# speedup vs baseline: 80.5196x; 63.5532x over previous
import functools

import jax
import jax.numpy as jnp
from jax import lax
from jax.experimental import pallas as pl
from jax.experimental.pallas import tpu as pltpu
from jax.experimental.pallas import tpu_sc as plsc

EPS = 1e-4
ALPHA = 4.0
N = 50000
NP = 800000
PS = 3
D = 128
H = 32

NW = 32
CH = 128
CHUNKS = 196
PPW = CH * CHUNKS
NPP = NW * PPW
N_POOL = 50048
RPT = N_POOL // 16
BN = 2000
NB = N // BN


def _tc1_body(f_ref, w_ref, e_ref, lt_ref):
    w2 = w_ref[...].reshape(PS * D, H)
    colnorm = jnp.maximum(jnp.sqrt(jnp.sum(w2 * w2, axis=0, keepdims=True)), EPS)
    wn = w2 / colnorm
    f = f_ref[...]
    inv = lax.rsqrt(jnp.maximum(jnp.sum(f * f, axis=1, keepdims=True), EPS * EPS))
    fn = f * inv
    wj = wn.reshape(PS, D, H)
    for j in range(PS):
        e_ref[j] = jnp.dot(fn, wj[j], preferred_element_type=jnp.float32) * (1.0 / PS)

    @pl.when(pl.program_id(0) == 0)
    def _():
        gram = jnp.exp(
            ALPHA * (lax.dot_general(wn, wn, (((0,), (0,)), ((), ())),
                                     preferred_element_type=jnp.float32) - 1.0))
        c = jnp.sqrt(jnp.sum(gram * gram))
        eye = jnp.eye(H, dtype=jnp.float32)
        y = gram * (1.0 / c)
        z = eye
        for _ in range(24):
            t = 1.5 * eye - 0.5 * jnp.dot(z, y, preferred_element_type=jnp.float32)
            y = jnp.dot(y, t, preferred_element_type=jnp.float32)
            z = jnp.dot(t, z, preferred_element_type=jnp.float32)
        lt_ref[...] = z * lax.rsqrt(c)


_tc1 = pl.pallas_call(
    _tc1_body,
    grid=(NB,),
    in_specs=[
        pl.BlockSpec((BN, D), lambda i: (i, 0)),
        pl.BlockSpec((PS, D, H), lambda i: (0, 0, 0)),
    ],
    out_specs=[
        pl.BlockSpec((PS, BN, H), lambda i: (0, i, 0)),
        pl.BlockSpec((H, H), lambda i: (0, 0)),
    ],
    out_shape=[
        jax.ShapeDtypeStruct((PS, N, H), jnp.float32),
        jax.ShapeDtypeStruct((H, H), jnp.float32),
    ],
)


def _sc_body(e_hbm, g0_hbm, g1_hbm, g2_hbm, seg_hbm, out_hbm,
             idx0, idx1, idx2, segv, r0, r1, r2, pooled, sem):
    c = lax.axis_index("c")
    s = lax.axis_index("s")
    wid = s * 2 + c

    def zero_row(p, _):
        r0[p, pl.ds(0, 16)] = jnp.zeros((16,), jnp.float32)
        r0[p, pl.ds(16, 16)] = jnp.zeros((16,), jnp.float32)
        return 0

    lax.fori_loop(0, CH, zero_row, 0)

    def zero_pool(i, _):
        pltpu.sync_copy(r0, pooled.at[pl.ds(s * RPT + i * CH, CH)])
        return 0

    lax.fori_loop(0, RPT // CH, zero_pool, 0)
    pltpu.sync_copy(r0.at[pl.ds(0, RPT - (RPT // CH) * CH)],
                    pooled.at[pl.ds(s * RPT + (RPT // CH) * CH,
                                    RPT - (RPT // CH) * CH)])
    plsc.subcore_barrier()

    def chunk(k, _):
        base = wid * PPW + k * CH
        pltpu.sync_copy(g0_hbm.at[pl.ds(base, CH)], idx0)
        pltpu.sync_copy(g1_hbm.at[pl.ds(base, CH)], idx1)
        pltpu.sync_copy(g2_hbm.at[pl.ds(base, CH)], idx2)
        pltpu.sync_copy(seg_hbm.at[pl.ds(base, CH)], segv)
        cp0 = pltpu.async_copy(e_hbm.at[idx0], r0, sem)
        cp1 = pltpu.async_copy(e_hbm.at[idx1], r1, sem)
        cp2 = pltpu.async_copy(e_hbm.at[idx2], r2, sem)
        cp0.wait()
        cp1.wait()
        cp2.wait()

        def compute(p, _):
            for h in (0, 16):
                v = r0[p, pl.ds(h, 16)] + r1[p, pl.ds(h, 16)] + r2[p, pl.ds(h, 16)]
                r0[p, pl.ds(h, 16)] = jnp.exp(ALPHA * v - ALPHA)
            return 0

        lax.fori_loop(0, CH, compute, 0)
        pltpu.sync_copy(r0, pooled.at[segv], add=True)
        return 0

    lax.fori_loop(0, CHUNKS, chunk, 0)
    plsc.subcore_barrier()
    pltpu.sync_copy(pooled.at[pl.ds(s * RPT, RPT)],
                    out_hbm.at[c, pl.ds(s * RPT, RPT)])


_sc_pool = functools.partial(
    pl.kernel,
    mesh=plsc.VectorSubcoreMesh(core_axis_name="c", subcore_axis_name="s"),
    compiler_params=pltpu.CompilerParams(use_tc_tiling_on_sc=False),
    out_type=jax.ShapeDtypeStruct((2, N_POOL, H), jnp.float32),
    scratch_types=[
        pltpu.VMEM((CH,), jnp.int32),
        pltpu.VMEM((CH,), jnp.int32),
        pltpu.VMEM((CH,), jnp.int32),
        pltpu.VMEM((CH,), jnp.int32),
        pltpu.VMEM((CH, H), jnp.float32),
        pltpu.VMEM((CH, H), jnp.float32),
        pltpu.VMEM((CH, H), jnp.float32),
        pltpu.VMEM_SHARED((N_POOL, H), jnp.float32),
        pltpu.SemaphoreType.DMA,
    ],
)(_sc_body)


def _tc2_body(p_ref, ks_ref, lt_ref, o_ref):
    p = p_ref[0] + p_ref[1]
    cnt = jnp.maximum(ks_ref[...], 1).astype(jnp.float32)
    o_ref[...] = jnp.dot(p / cnt, lt_ref[...], preferred_element_type=jnp.float32)


_tc2 = pl.pallas_call(
    _tc2_body,
    grid=(NB,),
    in_specs=[
        pl.BlockSpec((2, BN, H), lambda i: (0, i, 0)),
        pl.BlockSpec((BN, 1), lambda i: (i, 0)),
        pl.BlockSpec((H, H), lambda i: (0, 0)),
    ],
    out_specs=pl.BlockSpec((BN, H), lambda i: (i, 0)),
    out_shape=jax.ShapeDtypeStruct((N, H), jnp.float32),
)


@jax.jit
def kernel(features, paths_indices, kernel_size, weight):
    e3, lintrans = _tc1(features, weight)
    e_flat = e3.reshape(PS * N, H)

    cum = jnp.cumsum(kernel_size)
    z = jnp.zeros((NP,), jnp.int32).at[cum].add(1, mode='drop')
    seg = jnp.cumsum(z)
    segp = jnp.concatenate([seg, jnp.full((NPP - NP,), N, jnp.int32)])
    pad = jnp.zeros((NPP - NP,), jnp.int32)
    gs = [jnp.concatenate([paths_indices[:, j] + j * N, pad]) for j in range(PS)]

    pooled2 = _sc_pool(e_flat, gs[0], gs[1], gs[2], segp)[:, :N, :]
    return _tc2(pooled2, kernel_size.reshape(N, 1), lintrans)


# speedup vs baseline: 150.8204x; 1.8731x over previous
import functools

import jax
import jax.numpy as jnp
from jax import lax
from jax.experimental import pallas as pl
from jax.experimental.pallas import tpu as pltpu
from jax.experimental.pallas import tpu_sc as plsc

EPS = 1e-4
ALPHA = 4.0
N = 50000
NP = 800000
PS = 3
D = 128
H = 32

NW = 32
CH = 128
CHUNKS = 196
PPW = CH * CHUNKS
NPP = NW * PPW
N_POOL = 50048
RPT = N_POOL // 16
BN = 2000
NB = N // BN


def _tc1_body(f_ref, w_ref, e_ref, lt_ref):
    w2 = w_ref[...].reshape(PS * D, H)
    colnorm = jnp.maximum(jnp.sqrt(jnp.sum(w2 * w2, axis=0, keepdims=True)), EPS)
    wn = w2 / colnorm
    f = f_ref[...]
    inv = lax.rsqrt(jnp.maximum(jnp.sum(f * f, axis=1, keepdims=True), EPS * EPS))
    fn = f * inv
    wj = wn.reshape(PS, D, H)
    for j in range(PS):
        e_ref[j] = jnp.dot(fn, wj[j], preferred_element_type=jnp.float32) * (1.0 / PS)

    @pl.when(pl.program_id(0) == 0)
    def _():
        gram = jnp.exp(
            ALPHA * (lax.dot_general(wn, wn, (((0,), (0,)), ((), ())),
                                     preferred_element_type=jnp.float32) - 1.0))
        c = jnp.sqrt(jnp.sum(gram * gram))
        eye = jnp.eye(H, dtype=jnp.float32)
        y = gram * (1.0 / c)
        z = eye
        for _ in range(24):
            t = 1.5 * eye - 0.5 * jnp.dot(z, y, preferred_element_type=jnp.float32)
            y = jnp.dot(y, t, preferred_element_type=jnp.float32)
            z = jnp.dot(t, z, preferred_element_type=jnp.float32)
        lt_ref[...] = z * lax.rsqrt(c)


_tc1 = pl.pallas_call(
    _tc1_body,
    grid=(NB,),
    in_specs=[
        pl.BlockSpec((BN, D), lambda i: (i, 0)),
        pl.BlockSpec((PS, D, H), lambda i: (0, 0, 0)),
    ],
    out_specs=[
        pl.BlockSpec((PS, BN, H), lambda i: (0, i, 0)),
        pl.BlockSpec((H, H), lambda i: (0, 0)),
    ],
    out_shape=[
        jax.ShapeDtypeStruct((PS, N, H), jnp.float32),
        jax.ShapeDtypeStruct((H, H), jnp.float32),
    ],
)


def _sc_body(e_hbm, g4_hbm, out_hbm,
             b0, b1, r0, r1, pooled, sem_i, sem_g):
    c = lax.axis_index("c")
    s = lax.axis_index("s")
    wid = s * 2 + c
    cbase = wid * CHUNKS

    def zero_row(p, _):
        for h in (0, 16):
            r0[0, p, pl.ds(h, 16)] = jnp.zeros((16,), jnp.float32)
        return 0

    lax.fori_loop(0, CH, zero_row, 0)

    def zero_pool(i, _):
        pltpu.sync_copy(r0.at[0], pooled.at[pl.ds(s * RPT + i * CH, CH)])
        return 0

    lax.fori_loop(0, RPT // CH, zero_pool, 0)
    pltpu.sync_copy(r0.at[0, pl.ds(0, RPT - (RPT // CH) * CH)],
                    pooled.at[pl.ds(s * RPT + (RPT // CH) * CH,
                                    RPT - (RPT // CH) * CH)])
    plsc.subcore_barrier()

    def fire_gathers(b, r):
        for j in range(PS):
            pltpu.async_copy(e_hbm.at[b.at[j]], r.at[j], sem_g)

    def drain_gathers(r):
        for j in range(PS):
            pltpu.make_async_copy(e_hbm.at[pl.ds(0, CH)], r.at[j], sem_g).wait()

    def compute(r):
        def body(pb, _):
            for u in range(4):
                p = pb * 4 + u
                for h in (0, 16):
                    v = (r[0, p, pl.ds(h, 16)] + r[1, p, pl.ds(h, 16)]
                         + r[2, p, pl.ds(h, 16)])
                    r[0, p, pl.ds(h, 16)] = jnp.exp(ALPHA * v - ALPHA)
            return 0

        lax.fori_loop(0, CH // 4, body, 0)

    pltpu.sync_copy(g4_hbm.at[cbase], b0)
    fire_gathers(b0, r0)
    pltpu.async_copy(g4_hbm.at[cbase + 1], b1, sem_i)

    def step(i, _):
        for u, (bc, rc, bn, rn) in ((0, (b0, r0, b1, r1)),
                                    (1, (b1, r1, b0, r0))):
            m = 2 * i + u
            drain_gathers(rc)

            @pl.when(m + 1 < CHUNKS)
            def _():
                pltpu.make_async_copy(g4_hbm.at[cbase], bn, sem_i).wait()
                fire_gathers(bn, rn)

            compute(rc)
            pltpu.sync_copy(rc.at[0], pooled.at[bc.at[PS]], add=True)

            @pl.when(m + 2 < CHUNKS)
            def _():
                pltpu.async_copy(g4_hbm.at[cbase + m + 2], bc, sem_i)

        return 0

    lax.fori_loop(0, CHUNKS // 2, step, 0)
    plsc.subcore_barrier()
    pltpu.sync_copy(pooled.at[pl.ds(s * RPT, RPT)],
                    out_hbm.at[c, pl.ds(s * RPT, RPT)])


_sc_pool = functools.partial(
    pl.kernel,
    mesh=plsc.VectorSubcoreMesh(core_axis_name="c", subcore_axis_name="s"),
    compiler_params=pltpu.CompilerParams(use_tc_tiling_on_sc=False),
    out_type=jax.ShapeDtypeStruct((2, N_POOL, H), jnp.float32),
    scratch_types=[
        pltpu.VMEM((PS + 1, CH), jnp.int32),
        pltpu.VMEM((PS + 1, CH), jnp.int32),
        pltpu.VMEM((PS, CH, H), jnp.float32),
        pltpu.VMEM((PS, CH, H), jnp.float32),
        pltpu.VMEM_SHARED((N_POOL, H), jnp.float32),
        pltpu.SemaphoreType.DMA,
        pltpu.SemaphoreType.DMA,
    ],
)(_sc_body)


def _tc2_body(p_ref, ks_ref, lt_ref, o_ref):
    p = p_ref[0] + p_ref[1]
    cnt = jnp.maximum(ks_ref[...], 1).astype(jnp.float32)
    o_ref[...] = jnp.dot(p / cnt, lt_ref[...], preferred_element_type=jnp.float32)


_tc2 = pl.pallas_call(
    _tc2_body,
    grid=(NB,),
    in_specs=[
        pl.BlockSpec((2, BN, H), lambda i: (0, i, 0)),
        pl.BlockSpec((BN, 1), lambda i: (i, 0)),
        pl.BlockSpec((H, H), lambda i: (0, 0)),
    ],
    out_specs=pl.BlockSpec((BN, H), lambda i: (i, 0)),
    out_shape=jax.ShapeDtypeStruct((N, H), jnp.float32),
)


@jax.jit
def kernel(features, paths_indices, kernel_size, weight):
    e3, lintrans = _tc1(features, weight)
    e_flat = e3.reshape(PS * N, H)

    cum = jnp.cumsum(kernel_size)
    z = jnp.zeros((NP,), jnp.int32).at[cum].add(1, mode='drop')
    seg = jnp.cumsum(z)
    segp = jnp.concatenate([seg, jnp.full((NPP - NP,), N, jnp.int32)])
    pad = jnp.zeros((NPP - NP,), jnp.int32)
    gs = [jnp.concatenate([paths_indices[:, j] + j * N, pad]) for j in range(PS)]
    g4 = jnp.stack(gs + [segp], axis=0).reshape(PS + 1, NPP // CH, CH)
    g4 = g4.transpose(1, 0, 2)

    pooled2 = _sc_pool(e_flat, g4)[:, :N, :]
    return _tc2(pooled2, kernel_size.reshape(N, 1), lintrans)


# speedup vs baseline: 159.5369x; 1.0578x over previous
import functools

import jax
import jax.numpy as jnp
from jax import lax
from jax.experimental import pallas as pl
from jax.experimental.pallas import tpu as pltpu
from jax.experimental.pallas import tpu_sc as plsc

EPS = 1e-4
ALPHA = 4.0
N = 50000
NP = 800000
PS = 3
D = 128
H = 32

NW = 32
CH = 128
CHUNKS = 196
PPW = CH * CHUNKS
NPP = NW * PPW
N_POOL = 50048
RPT = N_POOL // 16
BN = 5000
NB = N // BN


def _tc1_body(f_ref, w_ref, e_ref, lt_ref):
    w2 = w_ref[...].reshape(PS * D, H)
    colnorm = jnp.maximum(jnp.sqrt(jnp.sum(w2 * w2, axis=0, keepdims=True)), EPS)
    wn = w2 / colnorm
    f = f_ref[...]
    inv = lax.rsqrt(jnp.maximum(jnp.sum(f * f, axis=1, keepdims=True), EPS * EPS))
    fn = f * inv
    wj = wn.reshape(PS, D, H)
    for j in range(PS):
        e_ref[j] = jnp.dot(fn, wj[j], preferred_element_type=jnp.float32) * (1.0 / PS)

    @pl.when(pl.program_id(0) == 0)
    def _():
        gram = jnp.exp(
            ALPHA * (lax.dot_general(wn, wn, (((0,), (0,)), ((), ())),
                                     preferred_element_type=jnp.float32) - 1.0))
        c = jnp.sqrt(jnp.sum(gram * gram))
        eye = jnp.eye(H, dtype=jnp.float32)
        y = gram * (1.0 / c)
        z = eye
        for _ in range(24):
            t = 1.5 * eye - 0.5 * jnp.dot(z, y, preferred_element_type=jnp.float32)
            y = jnp.dot(y, t, preferred_element_type=jnp.float32)
            z = jnp.dot(t, z, preferred_element_type=jnp.float32)
        lt_ref[...] = z * lax.rsqrt(c)


_tc1 = pl.pallas_call(
    _tc1_body,
    grid=(NB,),
    in_specs=[
        pl.BlockSpec((BN, D), lambda i: (i, 0)),
        pl.BlockSpec((PS, D, H), lambda i: (0, 0, 0)),
    ],
    out_specs=[
        pl.BlockSpec((PS, BN, H), lambda i: (0, i, 0)),
        pl.BlockSpec((H, H), lambda i: (0, 0)),
    ],
    out_shape=[
        jax.ShapeDtypeStruct((PS, N, H), jnp.float32),
        jax.ShapeDtypeStruct((H, H), jnp.float32),
    ],
)


def _sc_body(e_hbm, g4_hbm, out_hbm,
             b0, b1, r0, r1, pooled, sem_i, sg0, sg1):
    c = lax.axis_index("c")
    s = lax.axis_index("s")
    wid = s * 2 + c
    cbase = wid * CHUNKS

    def zero_row(p, _):
        for h in (0, 16):
            r0[0, p, pl.ds(h, 16)] = jnp.zeros((16,), jnp.float32)
        return 0

    lax.fori_loop(0, CH, zero_row, 0)

    nfull = RPT // CH
    rem = RPT - nfull * CH

    def zero_pool(i, _):
        pltpu.async_copy(r0.at[0], pooled.at[pl.ds(s * RPT + i * CH, CH)], sg1)
        return 0

    lax.fori_loop(0, nfull, zero_pool, 0)
    pltpu.async_copy(r0.at[0, pl.ds(0, rem)],
                     pooled.at[pl.ds(s * RPT + nfull * CH, rem)], sg0)

    def zero_wait(i, _):
        pltpu.make_async_copy(r0.at[0], pooled.at[pl.ds(s * RPT, CH)], sg1).wait()
        return 0

    lax.fori_loop(0, nfull, zero_wait, 0)
    pltpu.make_async_copy(r0.at[0, pl.ds(0, rem)],
                          pooled.at[pl.ds(s * RPT, rem)], sg0).wait()
    plsc.subcore_barrier()

    def fire_gathers(b, r, sg):
        for j in range(PS):
            pltpu.async_copy(e_hbm.at[b.at[j]], r.at[j], sg)

    def drain_gathers(r, sg):
        for j in range(PS):
            pltpu.make_async_copy(e_hbm.at[pl.ds(0, CH)], r.at[j], sg).wait()

    def compute(r):
        def body(pb, _):
            for u in range(4):
                p = pb * 4 + u
                for h in (0, 16):
                    v = (r[0, p, pl.ds(h, 16)] + r[1, p, pl.ds(h, 16)]
                         + r[2, p, pl.ds(h, 16)])
                    r[0, p, pl.ds(h, 16)] = jnp.exp(ALPHA * v - ALPHA)
            return 0

        lax.fori_loop(0, CH // 4, body, 0)

    pltpu.sync_copy(g4_hbm.at[cbase], b0)
    fire_gathers(b0, r0, sg0)
    pltpu.async_copy(g4_hbm.at[cbase + 1], b1, sem_i)

    def step(i, _):
        for u, bc, rc, sgc, bn, rn, sgn in ((0, b0, r0, sg0, b1, r1, sg1),
                                            (1, b1, r1, sg1, b0, r0, sg0)):
            m = 2 * i + u
            drain_gathers(rc, sgc)

            @pl.when(m + 1 < CHUNKS)
            def _():
                pltpu.make_async_copy(g4_hbm.at[cbase], bn, sem_i).wait()
                fire_gathers(bn, rn, sgn)

            compute(rc)
            pltpu.sync_copy(rc.at[0], pooled.at[bc.at[PS]], add=True)

            @pl.when(m + 2 < CHUNKS)
            def _():
                pltpu.async_copy(g4_hbm.at[cbase + m + 2], bc, sem_i)

        return 0

    lax.fori_loop(0, CHUNKS // 2, step, 0)
    plsc.subcore_barrier()
    pltpu.sync_copy(pooled.at[pl.ds(s * RPT, RPT)],
                    out_hbm.at[c, pl.ds(s * RPT, RPT)])


_sc_pool = functools.partial(
    pl.kernel,
    mesh=plsc.VectorSubcoreMesh(core_axis_name="c", subcore_axis_name="s"),
    compiler_params=pltpu.CompilerParams(use_tc_tiling_on_sc=False),
    out_type=jax.ShapeDtypeStruct((2, N_POOL, H), jnp.float32),
    scratch_types=[
        pltpu.VMEM((PS + 1, CH), jnp.int32),
        pltpu.VMEM((PS + 1, CH), jnp.int32),
        pltpu.VMEM((PS, CH, H), jnp.float32),
        pltpu.VMEM((PS, CH, H), jnp.float32),
        pltpu.VMEM_SHARED((N_POOL, H), jnp.float32),
        pltpu.SemaphoreType.DMA,
        pltpu.SemaphoreType.DMA,
        pltpu.SemaphoreType.DMA,
    ],
)(_sc_body)


def _tc2_body(p_ref, ks_ref, lt_ref, o_ref):
    p = p_ref[0] + p_ref[1]
    cnt = jnp.maximum(ks_ref[...], 1).astype(jnp.float32)
    o_ref[...] = jnp.dot(p / cnt, lt_ref[...], preferred_element_type=jnp.float32)


_tc2 = pl.pallas_call(
    _tc2_body,
    grid=(NB,),
    in_specs=[
        pl.BlockSpec((2, BN, H), lambda i: (0, i, 0)),
        pl.BlockSpec((BN, 1), lambda i: (i, 0)),
        pl.BlockSpec((H, H), lambda i: (0, 0)),
    ],
    out_specs=pl.BlockSpec((BN, H), lambda i: (i, 0)),
    out_shape=jax.ShapeDtypeStruct((N, H), jnp.float32),
)


@jax.jit
def kernel(features, paths_indices, kernel_size, weight):
    e3, lintrans = _tc1(features, weight)
    e_flat = e3.reshape(PS * N, H)

    cum = jnp.cumsum(kernel_size)
    z = jnp.zeros((NP,), jnp.int32).at[cum].add(1, mode='drop')
    seg = jnp.cumsum(z)
    segp = jnp.concatenate([seg, jnp.full((NPP - NP,), N, jnp.int32)])
    pad = jnp.zeros((NPP - NP,), jnp.int32)
    gs = [jnp.concatenate([paths_indices[:, j] + j * N, pad]) for j in range(PS)]
    g4 = jnp.stack(gs + [segp], axis=0).reshape(PS + 1, NPP // CH, CH)
    g4 = g4.transpose(1, 0, 2)

    pooled2 = _sc_pool(e_flat, g4)
    return _tc2(pooled2, kernel_size.reshape(N, 1), lintrans)
